# baseline (speedup 1.0000x reference)
import functools

import jax
import jax.numpy as jnp
from jax import lax
from jax.experimental import pallas as pl
from jax.experimental.pallas import tpu as pltpu
from jax.experimental.pallas import tpu_sc as plsc

N = 10000
E = 320000
D = 128
H = 128

NC = 2
NS = 16
L = 16
NW = NC * NS
EW = E // NW
K = 80
NCH = EW // K
NPAD = 10240
SEG = NPAD // NS

_mesh = plsc.VectorSubcoreMesh(core_axis_name="c", subcore_axis_name="s",
                               num_cores=NC, num_subcores=NS)


def _bitrev_perm():
    lane = lax.iota(jnp.int32, L)
    return (((lane & 1) << 3) | ((lane & 2) << 1)
            | ((lane & 4) >> 1) | ((lane & 8) >> 3))


def _merge_tree(accs, lane):
    cur = accs
    for s in (8, 4, 2, 1):
        perm = jnp.bitwise_xor(lane, s)
        nxt = []
        for i in range(len(cur) // 2):
            a, b = cur[2 * i], cur[2 * i + 1]
            pa = a + a[perm]
            pb = b + b[perm]
            nxt.append(jnp.where((lane & s) == 0, pa, pb))
        cur = nxt
    return cur[0][_bitrev_perm()]


def _copy_chunk(src_ref, src_off, dst_ref):
    for i in range(K // L):
        dst_ref[pl.ds(i * L, L)] = src_ref[pl.ds(src_off + i * L, L)]


BN = 1000


def _pre_body(h_ref, s_ref, w1t_ref, w1b_ref, a_ref, b_ref):
    a_ref[...] = jnp.dot(h_ref[...], w1t_ref[...],
                         preferred_element_type=jnp.float32)
    b_ref[...] = jnp.dot(s_ref[...], w1b_ref[...],
                         preferred_element_type=jnp.float32)


_pre = pl.pallas_call(
    _pre_body,
    grid=(N // BN,),
    in_specs=[pl.BlockSpec((BN, D), lambda i: (i, 0)),
              pl.BlockSpec((BN, D), lambda i: (i, 0)),
              pl.BlockSpec((D, H), lambda i: (0, 0)),
              pl.BlockSpec((D, H), lambda i: (0, 0))],
    out_specs=[pl.BlockSpec((BN, H), lambda i: (i, 0)),
               pl.BlockSpec((BN, H), lambda i: (i, 0))],
    out_shape=[jax.ShapeDtypeStruct((N, H), jnp.float32),
               jax.ShapeDtypeStruct((N, H), jnp.float32)],
)


def _edge1_body(a_hbm, b_hbm, src_hbm, dst_hbm, ew_hbm, w2_hbm,
                p_hbm, dpart_hbm,
                bufA0, bufA1, bufB0, bufB1, srcall, dstall, ewall,
                srcb0, srcb1, dstb0, dstb1,
                pb, w2v, zb, dsh, sA0, sA1, sB0, sB1):
    cid = lax.axis_index("c")
    sid = lax.axis_index("s")
    wid = cid * NS + sid
    ebase = wid * EW
    pltpu.sync_copy(w2_hbm, w2v)
    pltpu.sync_copy(src_hbm.at[pl.ds(ebase, EW)], srcall)
    pltpu.sync_copy(dst_hbm.at[pl.ds(ebase, EW)], dstall)
    pltpu.sync_copy(ew_hbm.at[pl.ds(ebase, EW)], ewall)

    def zb_init(i, c):
        zb[pl.ds(i * L, L)] = jnp.zeros((L,), jnp.float32)
        return c
    lax.fori_loop(0, SEG // L, zb_init, 0)
    pltpu.sync_copy(zb, dsh.at[pl.ds(sid * SEG, SEG)])
    plsc.subcore_barrier()

    lane = lax.iota(jnp.int32, L)
    bufsA = (bufA0, bufA1)
    bufsB = (bufB0, bufB1)
    srcbs = (srcb0, srcb1)
    dstbs = (dstb0, dstb1)
    semsA = (sA0, sA1)
    semsB = (sB0, sB1)

    def start(c, b):
        _copy_chunk(srcall, c * K, srcbs[b])
        _copy_chunk(dstall, c * K, dstbs[b])
        pltpu.async_copy(a_hbm.at[srcbs[b]], bufsA[b], semsA[b])
        pltpu.async_copy(b_hbm.at[dstbs[b]], bufsB[b], semsB[b])

    def wait(b):
        pltpu.make_async_copy(a_hbm.at[srcbs[b]], bufsA[b], semsA[b]).wait()
        pltpu.make_async_copy(b_hbm.at[dstbs[b]], bufsB[b], semsB[b]).wait()

    perms = [jnp.bitwise_xor(lane, s) for s in (8, 4, 2, 1)]

    def compute(c, b):
        bufA, bufB = bufsA[b], bufsB[b]

        def group(g, carry2):
            gbase = g * L

            def ebody(e, rawv):
                row = gbase + e

                def jbody(jo, acc):
                    x = (bufA[row, pl.ds(jo * L, L)]
                         + bufB[row, pl.ds(jo * L, L)])
                    e2 = jnp.exp(x + x)
                    t = 1.0 - 2.0 / (e2 + 1.0)
                    return acc + t * w2v[pl.ds(jo * L, L)]

                acc = lax.fori_loop(0, H // L, jbody,
                                    jnp.zeros((L,), jnp.float32), unroll=8)
                for pm in perms:
                    acc = acc + acc[pm]
                return jnp.where(lane == e, acc, rawv)

            rawv = lax.fori_loop(0, L, ebody, jnp.zeros((L,), jnp.float32))
            pv = jnp.exp(rawv * ewall[pl.ds(c * K + gbase, L)])
            pb[pl.ds(gbase, L)] = pv
            return carry2
        lax.fori_loop(0, K // L, group, 0)
        pltpu.sync_copy(pb, p_hbm.at[pl.ds(ebase + c * K, K)])
        pltpu.sync_copy(pb, dsh.at[dstbs[b]], add=True)

    start(0, 0)
    start(1, 1)

    def outer(c2, carry):
        for b in range(2):
            c = c2 * 2 + b
            wait(b)
            compute(c, b)

            @pl.when(c + 2 < NCH)
            def _():
                start(c + 2, b)
        return carry
    lax.fori_loop(0, NCH // 2, outer, 0)
    wait(0)
    compute(NCH - 1, 0)

    plsc.subcore_barrier()
    pltpu.sync_copy(dsh.at[pl.ds(sid * SEG, SEG)],
                    dpart_hbm.at[cid, pl.ds(sid * SEG, SEG)])


_edge1 = functools.partial(
    pl.kernel,
    out_type=[jax.ShapeDtypeStruct((E,), jnp.float32),
              jax.ShapeDtypeStruct((NC, NPAD), jnp.float32)],
    mesh=_mesh,
    scratch_types=[
        pltpu.VMEM((K, H), jnp.float32),
        pltpu.VMEM((K, H), jnp.float32),
        pltpu.VMEM((K, H), jnp.float32),
        pltpu.VMEM((K, H), jnp.float32),
        pltpu.VMEM((EW,), jnp.int32),
        pltpu.VMEM((EW,), jnp.int32),
        pltpu.VMEM((EW,), jnp.float32),
        pltpu.VMEM((K,), jnp.int32),
        pltpu.VMEM((K,), jnp.int32),
        pltpu.VMEM((K,), jnp.int32),
        pltpu.VMEM((K,), jnp.int32),
        pltpu.VMEM((K,), jnp.float32),
        pltpu.VMEM((H,), jnp.float32),
        pltpu.VMEM((SEG,), jnp.float32),
        pltpu.VMEM_SHARED((NPAD,), jnp.float32),
        pltpu.SemaphoreType.DMA,
        pltpu.SemaphoreType.DMA,
        pltpu.SemaphoreType.DMA,
        pltpu.SemaphoreType.DMA,
    ],
)(_edge1_body)


def _edge2_body(hsrc_hbm, src_hbm, dst_hbm, p_hbm, dp0_hbm, dp1_hbm,
                attn_hbm, hg_hbm,
                rb0, rb1, rb2, sc0,
                srcb0, srcb1, srcb2, srcb3, dstb0, dstb1, dstb2, dstb3,
                d0c0, d0c1, d0c2, d1c0, d1c1, d1c2, pc0, pc1, pc2, attnb,
                hg_sh, sG0, sG1, sG2, sI0, sI1, sI2, sI3):
    cid = lax.axis_index("c")
    sid = lax.axis_index("s")
    wid = cid * NS + sid
    ebase = wid * EW

    rbufs = (rb0, rb1, rb2)
    srcbs = (srcb0, srcb1, srcb2, srcb3)
    dstbs = (dstb0, dstb1, dstb2, dstb3)
    d0chs = (d0c0, d0c1, d0c2)
    d1chs = (d1c0, d1c1, d1c2)
    pchs = (pc0, pc1, pc2)
    semsG = (sG0, sG1, sG2)
    semsI = (sI0, sI1, sI2, sI3)

    def zrow(r, c):
        for j in range(H // L):
            rb0[r, pl.ds(j * L, L)] = jnp.zeros((L,), jnp.float32)
        return c
    lax.fori_loop(0, K, zrow, 0)
    for i in range(SEG // K):
        pltpu.sync_copy(rb0, hg_sh.at[pl.ds(sid * SEG + i * K, K)])
    plsc.subcore_barrier()

    def start_idx(c, b4):
        off = ebase + c * K
        pltpu.async_copy(src_hbm.at[pl.ds(off, K)], srcbs[b4], semsI[b4])
        pltpu.async_copy(dst_hbm.at[pl.ds(off, K)], dstbs[b4], semsI[b4])

    def wait_idx(c, b4):
        off = ebase + c * K
        pltpu.make_async_copy(src_hbm.at[pl.ds(off, K)], srcbs[b4],
                              semsI[b4]).wait()
        pltpu.make_async_copy(dst_hbm.at[pl.ds(off, K)], dstbs[b4],
                              semsI[b4]).wait()

    def start_gather(c, b3, b4):
        off = ebase + c * K
        pltpu.async_copy(hsrc_hbm.at[srcbs[b4]], rbufs[b3], semsG[b3])
        pltpu.async_copy(dp0_hbm.at[dstbs[b4]], d0chs[b3], semsG[b3])
        pltpu.async_copy(dp1_hbm.at[dstbs[b4]], d1chs[b3], semsG[b3])
        pltpu.async_copy(p_hbm.at[pl.ds(off, K)], pchs[b3], semsG[b3])

    def wait_gather(c, b3, b4):
        off = ebase + c * K
        pltpu.make_async_copy(hsrc_hbm.at[srcbs[b4]], rbufs[b3],
                              semsG[b3]).wait()
        pltpu.make_async_copy(dp0_hbm.at[dstbs[b4]], d0chs[b3],
                              semsG[b3]).wait()
        pltpu.make_async_copy(dp1_hbm.at[dstbs[b4]], d1chs[b3],
                              semsG[b3]).wait()
        pltpu.make_async_copy(p_hbm.at[pl.ds(off, K)], pchs[b3],
                              semsG[b3]).wait()

    def do_chunk(c, b3, b4, prefetch3, prefetch4):
        wait_gather(c, b3, b4)
        rowbuf = rbufs[b3]
        d0ch, d1ch, pch = d0chs[b3], d1chs[b3], pchs[b3]

        def group(g, carry2):
            gbase = g * L
            denv = d0ch[pl.ds(gbase, L)] + d1ch[pl.ds(gbase, L)]
            attnv = pch[pl.ds(gbase, L)] / denv
            attnb[pl.ds(gbase, L)] = attnv

            def ebody(e, c2):
                row = gbase + e
                av = attnv[jnp.full((L,), e, jnp.int32)]

                def jbody(jo, c3):
                    sc0[row, pl.ds(jo * L, L)] = (
                        rowbuf[row, pl.ds(jo * L, L)] * av)
                    return c3
                return lax.fori_loop(0, H // L, jbody, c2, unroll=8)
            return lax.fori_loop(0, L, ebody, carry2)
        lax.fori_loop(0, K // L, group, 0)
        pltpu.sync_copy(attnb, attn_hbm.at[pl.ds(ebase + c * K, K)])
        pltpu.sync_copy(sc0, hg_sh.at[dstbs[b4]], add=True)
        if prefetch3:
            wait_idx(c + 3, (b4 + 3) % 4)
            start_gather(c + 3, b3, (b4 + 3) % 4)
        if prefetch4:
            start_idx(c + 4, b4)

    for c in range(4):
        start_idx(c, c)
    for c in range(3):
        wait_idx(c, c)
        start_gather(c, c, c)

    def outer(c12, carry):
        for i in range(12):
            c = c12 * 12 + i
            do_chunk(c, i % 3, i % 4, True, True)
        return carry
    lax.fori_loop(0, NCH // 12, outer, 0)
    for c in range((NCH // 12) * 12, NCH):
        do_chunk(c, c % 3, c % 4, c + 3 < NCH, c + 4 < NCH)

    plsc.subcore_barrier()
    pltpu.sync_copy(hg_sh.at[pl.ds(sid * SEG, SEG)],
                    hg_hbm.at[cid, pl.ds(sid * SEG, SEG)])


_edge2 = functools.partial(
    pl.kernel,
    out_type=[jax.ShapeDtypeStruct((E,), jnp.float32),
              jax.ShapeDtypeStruct((NC, NPAD, H), jnp.float32)],
    mesh=_mesh,
    scratch_types=[
        pltpu.VMEM((K, H), jnp.float32),
        pltpu.VMEM((K, H), jnp.float32),
        pltpu.VMEM((K, H), jnp.float32),
        pltpu.VMEM((K, H), jnp.float32),
        pltpu.VMEM((K,), jnp.int32),
        pltpu.VMEM((K,), jnp.int32),
        pltpu.VMEM((K,), jnp.int32),
        pltpu.VMEM((K,), jnp.int32),
        pltpu.VMEM((K,), jnp.int32),
        pltpu.VMEM((K,), jnp.int32),
        pltpu.VMEM((K,), jnp.int32),
        pltpu.VMEM((K,), jnp.int32),
        pltpu.VMEM((K,), jnp.float32),
        pltpu.VMEM((K,), jnp.float32),
        pltpu.VMEM((K,), jnp.float32),
        pltpu.VMEM((K,), jnp.float32),
        pltpu.VMEM((K,), jnp.float32),
        pltpu.VMEM((K,), jnp.float32),
        pltpu.VMEM((K,), jnp.float32),
        pltpu.VMEM((K,), jnp.float32),
        pltpu.VMEM((K,), jnp.float32),
        pltpu.VMEM((K,), jnp.float32),
        pltpu.VMEM_SHARED((NPAD, H), jnp.float32),
        pltpu.SemaphoreType.DMA,
        pltpu.SemaphoreType.DMA,
        pltpu.SemaphoreType.DMA,
        pltpu.SemaphoreType.DMA,
        pltpu.SemaphoreType.DMA,
        pltpu.SemaphoreType.DMA,
        pltpu.SemaphoreType.DMA,
    ],
)(_edge2_body)


def _post_body(hd_ref, g0_ref, g1_ref, wt_ref, wb_ref, b_ref, gam_ref,
               bet_ref, o_ref):
    hd = hd_ref[...]
    hg = g0_ref[...] + g1_ref[...]
    proj = (jnp.dot(hd, wt_ref[...], preferred_element_type=jnp.float32)
            + jnp.dot(hg, wb_ref[...], preferred_element_type=jnp.float32)
            + b_ref[...])
    x = hd + proj
    mu = jnp.mean(x, axis=-1, keepdims=True)
    xc = x - mu
    var = jnp.mean(xc * xc, axis=-1, keepdims=True)
    xn = xc * lax.rsqrt(var + 1e-5)
    o_ref[...] = xn * gam_ref[...] + bet_ref[...]


_post = pl.pallas_call(
    _post_body,
    grid=(N // BN,),
    in_specs=[pl.BlockSpec((BN, D), lambda i: (i, 0)),
              pl.BlockSpec((BN, H), lambda i: (i, 0)),
              pl.BlockSpec((BN, H), lambda i: (i, 0)),
              pl.BlockSpec((D, D), lambda i: (0, 0)),
              pl.BlockSpec((H, D), lambda i: (0, 0)),
              pl.BlockSpec((1, D), lambda i: (0, 0)),
              pl.BlockSpec((1, D), lambda i: (0, 0)),
              pl.BlockSpec((1, D), lambda i: (0, 0))],
    out_specs=pl.BlockSpec((BN, D), lambda i: (i, 0)),
    out_shape=jax.ShapeDtypeStruct((N, D), jnp.float32),
)


def kernel(h_src, h_dst, session_embedding, edge_index, edge_weight,
           W_attn1, W_attn2, W_out, b_out, ln_gamma, ln_beta):
    src = edge_index[0]
    dst = edge_index[1]
    w2 = W_attn2[:, 0]
    A, B = _pre(h_src, session_embedding, W_attn1[:D], W_attn1[D:])
    p, dpart = _edge1(A, B, src, dst, edge_weight, w2)
    attn, hg = _edge2(h_src, src, dst, p, dpart[0], dpart[1])
    out = _post(h_dst, hg[0, :N], hg[1, :N], W_out[:D], W_out[D:],
                b_out.reshape(1, D), ln_gamma.reshape(1, D),
                ln_beta.reshape(1, D))
    return out, attn[:, None]


# speedup vs baseline: 1.1573x; 1.1573x over previous
import functools

import jax
import jax.numpy as jnp
from jax import lax
from jax.experimental import pallas as pl
from jax.experimental.pallas import tpu as pltpu
from jax.experimental.pallas import tpu_sc as plsc

N = 10000
E = 320000
D = 128
H = 128

NC = 2
NS = 16
L = 16
NW = NC * NS
EW = E // NW
K = 80
NCH = EW // K
NPAD = 10240
SEG = NPAD // NS

_mesh = plsc.VectorSubcoreMesh(core_axis_name="c", subcore_axis_name="s",
                               num_cores=NC, num_subcores=NS)


def _bitrev_perm():
    lane = lax.iota(jnp.int32, L)
    return (((lane & 1) << 3) | ((lane & 2) << 1)
            | ((lane & 4) >> 1) | ((lane & 8) >> 3))


def _merge_tree(accs, lane):
    cur = accs
    for s in (8, 4, 2, 1):
        perm = jnp.bitwise_xor(lane, s)
        nxt = []
        for i in range(len(cur) // 2):
            a, b = cur[2 * i], cur[2 * i + 1]
            pa = a + a[perm]
            pb = b + b[perm]
            nxt.append(jnp.where((lane & s) == 0, pa, pb))
        cur = nxt
    return cur[0][_bitrev_perm()]


def _copy_chunk(src_ref, src_off, dst_ref):
    for i in range(K // L):
        dst_ref[pl.ds(i * L, L)] = src_ref[pl.ds(src_off + i * L, L)]


BN = 1000


def _pre_body(h_ref, s_ref, w1t_ref, w1b_ref, a_ref, b_ref):
    a_ref[...] = jnp.dot(h_ref[...], w1t_ref[...],
                         preferred_element_type=jnp.float32)
    b_ref[...] = jnp.dot(s_ref[...], w1b_ref[...],
                         preferred_element_type=jnp.float32)


_pre = pl.pallas_call(
    _pre_body,
    grid=(N // BN,),
    in_specs=[pl.BlockSpec((BN, D), lambda i: (i, 0)),
              pl.BlockSpec((BN, D), lambda i: (i, 0)),
              pl.BlockSpec((D, H), lambda i: (0, 0)),
              pl.BlockSpec((D, H), lambda i: (0, 0))],
    out_specs=[pl.BlockSpec((BN, H), lambda i: (i, 0)),
               pl.BlockSpec((BN, H), lambda i: (i, 0))],
    out_shape=[jax.ShapeDtypeStruct((N, H), jnp.float32),
               jax.ShapeDtypeStruct((N, H), jnp.float32)],
)


def _edge1_body(a_hbm, b_hbm, src_hbm, dst_hbm, ew_hbm, w2_hbm,
                p_hbm, dpart_hbm,
                bufA0, bufA1, bufB0, bufB1, srcall, dstall, ewall,
                srcb0, srcb1, dstb0, dstb1,
                pb, w2v, zb, dsh, sA0, sA1, sB0, sB1):
    cid = lax.axis_index("c")
    sid = lax.axis_index("s")
    wid = cid * NS + sid
    ebase = wid * EW
    pltpu.sync_copy(w2_hbm, w2v)
    pltpu.sync_copy(src_hbm.at[pl.ds(ebase, EW)], srcall)
    pltpu.sync_copy(dst_hbm.at[pl.ds(ebase, EW)], dstall)
    pltpu.sync_copy(ew_hbm.at[pl.ds(ebase, EW)], ewall)

    def zb_init(i, c):
        zb[pl.ds(i * L, L)] = jnp.zeros((L,), jnp.float32)
        return c
    lax.fori_loop(0, SEG // L, zb_init, 0)
    pltpu.sync_copy(zb, dsh.at[pl.ds(sid * SEG, SEG)])
    plsc.subcore_barrier()

    lane = lax.iota(jnp.int32, L)
    bufsA = (bufA0, bufA1)
    bufsB = (bufB0, bufB1)
    srcbs = (srcb0, srcb1)
    dstbs = (dstb0, dstb1)
    semsA = (sA0, sA1)
    semsB = (sB0, sB1)

    def start(c, b):
        _copy_chunk(srcall, c * K, srcbs[b])
        _copy_chunk(dstall, c * K, dstbs[b])
        pltpu.async_copy(a_hbm.at[srcbs[b]], bufsA[b], semsA[b])
        pltpu.async_copy(b_hbm.at[dstbs[b]], bufsB[b], semsB[b])

    def wait(b):
        pltpu.make_async_copy(a_hbm.at[srcbs[b]], bufsA[b], semsA[b]).wait()
        pltpu.make_async_copy(b_hbm.at[dstbs[b]], bufsB[b], semsB[b]).wait()

    perms = [jnp.bitwise_xor(lane, s) for s in (8, 4, 2, 1)]

    def compute(c, b):
        bufA, bufB = bufsA[b], bufsB[b]

        def group(g, carry2):
            gbase = g * L

            def ebody(e, rawv):
                row = gbase + e

                def jbody(jo, acc):
                    x = (bufA[row, pl.ds(jo * L, L)]
                         + bufB[row, pl.ds(jo * L, L)])
                    e2 = jnp.exp(x + x)
                    t = 1.0 - 2.0 / (e2 + 1.0)
                    return acc + t * w2v[pl.ds(jo * L, L)]

                acc = lax.fori_loop(0, H // L, jbody,
                                    jnp.zeros((L,), jnp.float32), unroll=8)
                for pm in perms:
                    acc = acc + acc[pm]
                return jnp.where(lane == e, acc, rawv)

            rawv = lax.fori_loop(0, L, ebody, jnp.zeros((L,), jnp.float32),
                                 unroll=2)
            pv = jnp.exp(rawv * ewall[pl.ds(c * K + gbase, L)])
            pb[pl.ds(gbase, L)] = pv
            return carry2
        lax.fori_loop(0, K // L, group, 0)
        pltpu.sync_copy(pb, p_hbm.at[pl.ds(ebase + c * K, K)])
        pltpu.sync_copy(pb, dsh.at[dstbs[b]], add=True)

    start(0, 0)
    start(1, 1)

    def outer(c2, carry):
        for b in range(2):
            c = c2 * 2 + b
            wait(b)
            compute(c, b)

            @pl.when(c + 2 < NCH)
            def _():
                start(c + 2, b)
        return carry
    lax.fori_loop(0, NCH // 2, outer, 0)
    wait(0)
    compute(NCH - 1, 0)

    plsc.subcore_barrier()
    pltpu.sync_copy(dsh.at[pl.ds(sid * SEG, SEG)],
                    dpart_hbm.at[cid, pl.ds(sid * SEG, SEG)])


_edge1 = functools.partial(
    pl.kernel,
    out_type=[jax.ShapeDtypeStruct((E,), jnp.float32),
              jax.ShapeDtypeStruct((NC, NPAD), jnp.float32)],
    mesh=_mesh,
    scratch_types=[
        pltpu.VMEM((K, H), jnp.float32),
        pltpu.VMEM((K, H), jnp.float32),
        pltpu.VMEM((K, H), jnp.float32),
        pltpu.VMEM((K, H), jnp.float32),
        pltpu.VMEM((EW,), jnp.int32),
        pltpu.VMEM((EW,), jnp.int32),
        pltpu.VMEM((EW,), jnp.float32),
        pltpu.VMEM((K,), jnp.int32),
        pltpu.VMEM((K,), jnp.int32),
        pltpu.VMEM((K,), jnp.int32),
        pltpu.VMEM((K,), jnp.int32),
        pltpu.VMEM((K,), jnp.float32),
        pltpu.VMEM((H,), jnp.float32),
        pltpu.VMEM((SEG,), jnp.float32),
        pltpu.VMEM_SHARED((NPAD,), jnp.float32),
        pltpu.SemaphoreType.DMA,
        pltpu.SemaphoreType.DMA,
        pltpu.SemaphoreType.DMA,
        pltpu.SemaphoreType.DMA,
    ],
)(_edge1_body)


def _dsum_body(d_ref, o_ref):
    o_ref[...] = d_ref[0] + d_ref[1]


_dsum = pl.pallas_call(
    _dsum_body,
    grid=(1,),
    in_specs=[pl.BlockSpec((NC, 8, NPAD // 8), lambda i: (0, 0, 0))],
    out_specs=pl.BlockSpec((8, NPAD // 8), lambda i: (0, 0)),
    out_shape=jax.ShapeDtypeStruct((8, NPAD // 8), jnp.float32),
)


def _edge2_body(hsrc_hbm, src_hbm, dst_hbm, p_hbm, den_hbm,
                attn_hbm, hg_hbm,
                rowbuf, srcall, dstall, pall, srcb, dstb,
                dch, attnb, hg_sh):
    cid = lax.axis_index("c")
    sid = lax.axis_index("s")
    wid = cid * NS + sid
    ebase = wid * EW
    pltpu.sync_copy(src_hbm.at[pl.ds(ebase, EW)], srcall)
    pltpu.sync_copy(dst_hbm.at[pl.ds(ebase, EW)], dstall)
    pltpu.sync_copy(p_hbm.at[pl.ds(ebase, EW)], pall)

    def zrow(r, c):
        for j in range(H // L):
            rowbuf[r, pl.ds(j * L, L)] = jnp.zeros((L,), jnp.float32)
        return c
    lax.fori_loop(0, K, zrow, 0)
    for i in range(SEG // K):
        pltpu.sync_copy(rowbuf, hg_sh.at[pl.ds(sid * SEG + i * K, K)])
    plsc.subcore_barrier()

    def chunk(c, carry):
        _copy_chunk(srcall, c * K, srcb)
        _copy_chunk(dstall, c * K, dstb)
        pltpu.sync_copy(hsrc_hbm.at[srcb], rowbuf)
        pltpu.sync_copy(den_hbm.at[dstb], dch)

        def group(g, carry2):
            gbase = g * L
            attnv = pall[pl.ds(c * K + gbase, L)] / dch[pl.ds(gbase, L)]
            attnb[pl.ds(gbase, L)] = attnv

            def ebody(e, c2):
                row = gbase + e
                av = attnv[jnp.full((L,), e, jnp.int32)]

                def jbody(jo, c3):
                    rowbuf[row, pl.ds(jo * L, L)] = (
                        rowbuf[row, pl.ds(jo * L, L)] * av)
                    return c3
                return lax.fori_loop(0, H // L, jbody, c2, unroll=8)
            return lax.fori_loop(0, L, ebody, carry2)
        lax.fori_loop(0, K // L, group, 0)
        pltpu.sync_copy(attnb, attn_hbm.at[pl.ds(ebase + c * K, K)])
        pltpu.sync_copy(rowbuf, hg_sh.at[dstb], add=True)
        return carry
    lax.fori_loop(0, NCH, chunk, 0)

    plsc.subcore_barrier()
    pltpu.sync_copy(hg_sh.at[pl.ds(sid * SEG, SEG)],
                    hg_hbm.at[cid, pl.ds(sid * SEG, SEG)])


_edge2 = functools.partial(
    pl.kernel,
    out_type=[jax.ShapeDtypeStruct((E,), jnp.float32),
              jax.ShapeDtypeStruct((NC, NPAD, H), jnp.float32)],
    mesh=_mesh,
    scratch_types=[
        pltpu.VMEM((K, H), jnp.float32),
        pltpu.VMEM((EW,), jnp.int32),
        pltpu.VMEM((EW,), jnp.int32),
        pltpu.VMEM((EW,), jnp.float32),
        pltpu.VMEM((K,), jnp.int32),
        pltpu.VMEM((K,), jnp.int32),
        pltpu.VMEM((K,), jnp.float32),
        pltpu.VMEM((K,), jnp.float32),
        pltpu.VMEM_SHARED((NPAD, H), jnp.float32),
    ],
)(_edge2_body)


def _post_body(hd_ref, g0_ref, g1_ref, wt_ref, wb_ref, b_ref, gam_ref,
               bet_ref, o_ref):
    hd = hd_ref[...]
    hg = g0_ref[...] + g1_ref[...]
    proj = (jnp.dot(hd, wt_ref[...], preferred_element_type=jnp.float32)
            + jnp.dot(hg, wb_ref[...], preferred_element_type=jnp.float32)
            + b_ref[...])
    x = hd + proj
    mu = jnp.mean(x, axis=-1, keepdims=True)
    xc = x - mu
    var = jnp.mean(xc * xc, axis=-1, keepdims=True)
    xn = xc * lax.rsqrt(var + 1e-5)
    o_ref[...] = xn * gam_ref[...] + bet_ref[...]


_post = pl.pallas_call(
    _post_body,
    grid=(N // BN,),
    in_specs=[pl.BlockSpec((BN, D), lambda i: (i, 0)),
              pl.BlockSpec((BN, H), lambda i: (i, 0)),
              pl.BlockSpec((BN, H), lambda i: (i, 0)),
              pl.BlockSpec((D, D), lambda i: (0, 0)),
              pl.BlockSpec((H, D), lambda i: (0, 0)),
              pl.BlockSpec((1, D), lambda i: (0, 0)),
              pl.BlockSpec((1, D), lambda i: (0, 0)),
              pl.BlockSpec((1, D), lambda i: (0, 0))],
    out_specs=pl.BlockSpec((BN, D), lambda i: (i, 0)),
    out_shape=jax.ShapeDtypeStruct((N, D), jnp.float32),
)


def kernel(h_src, h_dst, session_embedding, edge_index, edge_weight,
           W_attn1, W_attn2, W_out, b_out, ln_gamma, ln_beta):
    src = edge_index[0]
    dst = edge_index[1]
    w2 = W_attn2[:, 0]
    A, B = _pre(h_src, session_embedding, W_attn1[:D], W_attn1[D:])
    p, dpart = _edge1(A, B, src, dst, edge_weight, w2)
    den = _dsum(dpart.reshape(NC, 8, NPAD // 8)).reshape(NPAD)
    attn, hg = _edge2(h_src, src, dst, p, den)
    out = _post(h_dst, hg[0, :N], hg[1, :N], W_out[:D], W_out[D:],
                b_out.reshape(1, D), ln_gamma.reshape(1, D),
                ln_beta.reshape(1, D))
    return out, attn[:, None]


# speedup vs baseline: 1.1690x; 1.0101x over previous
import functools

import jax
import jax.numpy as jnp
from jax import lax
from jax.experimental import pallas as pl
from jax.experimental.pallas import tpu as pltpu
from jax.experimental.pallas import tpu_sc as plsc

N = 10000
E = 320000
D = 128
H = 128

NC = 2
NS = 16
L = 16
NW = NC * NS
EW = E // NW
K = 80
NCH = EW // K
NPAD = 10240
SEG = NPAD // NS

_mesh = plsc.VectorSubcoreMesh(core_axis_name="c", subcore_axis_name="s",
                               num_cores=NC, num_subcores=NS)


def _bitrev_perm():
    lane = lax.iota(jnp.int32, L)
    return (((lane & 1) << 3) | ((lane & 2) << 1)
            | ((lane & 4) >> 1) | ((lane & 8) >> 3))


def _merge_tree(accs, lane):
    cur = accs
    for s in (8, 4, 2, 1):
        perm = jnp.bitwise_xor(lane, s)
        nxt = []
        for i in range(len(cur) // 2):
            a, b = cur[2 * i], cur[2 * i + 1]
            pa = a + a[perm]
            pb = b + b[perm]
            nxt.append(jnp.where((lane & s) == 0, pa, pb))
        cur = nxt
    return cur[0][_bitrev_perm()]


def _copy_chunk(src_ref, src_off, dst_ref):
    for i in range(K // L):
        dst_ref[pl.ds(i * L, L)] = src_ref[pl.ds(src_off + i * L, L)]


BN = 1000


def _pre_body(h_ref, s_ref, w1t_ref, w1b_ref, a_ref, b_ref):
    a_ref[...] = jnp.dot(h_ref[...], w1t_ref[...],
                         preferred_element_type=jnp.float32)
    b_ref[...] = jnp.dot(s_ref[...], w1b_ref[...],
                         preferred_element_type=jnp.float32)


_pre = pl.pallas_call(
    _pre_body,
    grid=(N // BN,),
    in_specs=[pl.BlockSpec((BN, D), lambda i: (i, 0)),
              pl.BlockSpec((BN, D), lambda i: (i, 0)),
              pl.BlockSpec((D, H), lambda i: (0, 0)),
              pl.BlockSpec((D, H), lambda i: (0, 0))],
    out_specs=[pl.BlockSpec((BN, H), lambda i: (i, 0)),
               pl.BlockSpec((BN, H), lambda i: (i, 0))],
    out_shape=[jax.ShapeDtypeStruct((N, H), jnp.float32),
               jax.ShapeDtypeStruct((N, H), jnp.float32)],
)


def _edge1_body(a_hbm, b_hbm, src_hbm, dst_hbm, ew_hbm, w2_hbm,
                p_hbm, dpart_hbm,
                bufA0, bufA1, bufB0, bufB1, srcall, dstall, ewall,
                srcb0, srcb1, dstb0, dstb1,
                pb, w2v, zb, dsh, sA0, sA1, sB0, sB1):
    cid = lax.axis_index("c")
    sid = lax.axis_index("s")
    wid = cid * NS + sid
    ebase = wid * EW
    pltpu.sync_copy(w2_hbm, w2v)
    pltpu.sync_copy(src_hbm.at[pl.ds(ebase, EW)], srcall)
    pltpu.sync_copy(dst_hbm.at[pl.ds(ebase, EW)], dstall)
    pltpu.sync_copy(ew_hbm.at[pl.ds(ebase, EW)], ewall)

    def zb_init(i, c):
        zb[pl.ds(i * L, L)] = jnp.zeros((L,), jnp.float32)
        return c
    lax.fori_loop(0, SEG // L, zb_init, 0)
    pltpu.sync_copy(zb, dsh.at[pl.ds(sid * SEG, SEG)])
    plsc.subcore_barrier()

    lane = lax.iota(jnp.int32, L)
    w2s = [w2v[pl.ds(j * L, L)] for j in range(H // L)]
    bufsA = (bufA0, bufA1)
    bufsB = (bufB0, bufB1)
    srcbs = (srcb0, srcb1)
    dstbs = (dstb0, dstb1)
    semsA = (sA0, sA1)
    semsB = (sB0, sB1)

    def start(c, b):
        _copy_chunk(srcall, c * K, srcbs[b])
        _copy_chunk(dstall, c * K, dstbs[b])
        pltpu.async_copy(a_hbm.at[srcbs[b]], bufsA[b], semsA[b])
        pltpu.async_copy(b_hbm.at[dstbs[b]], bufsB[b], semsB[b])

    def wait(b):
        pltpu.make_async_copy(a_hbm.at[srcbs[b]], bufsA[b], semsA[b]).wait()
        pltpu.make_async_copy(b_hbm.at[dstbs[b]], bufsB[b], semsB[b]).wait()

    perms = [jnp.bitwise_xor(lane, s) for s in (8, 4, 2, 1)]

    def compute(c, b):
        bufA, bufB = bufsA[b], bufsB[b]

        def group(g, carry2):
            gbase = g * L

            def ebody(e, rawv):
                row = gbase + e

                acc = jnp.zeros((L,), jnp.float32)
                for jo in range(H // L):
                    x = (bufA[row, pl.ds(jo * L, L)]
                         + bufB[row, pl.ds(jo * L, L)])
                    e2 = jnp.exp(x + x)
                    t = 1.0 - 2.0 / (e2 + 1.0)
                    acc = acc + t * w2s[jo]
                for pm in perms:
                    acc = acc + acc[pm]
                return jnp.where(lane == e, acc, rawv)

            rawv = lax.fori_loop(0, L, ebody, jnp.zeros((L,), jnp.float32),
                                 unroll=2)
            pv = jnp.exp(rawv * ewall[pl.ds(c * K + gbase, L)])
            pb[pl.ds(gbase, L)] = pv
            return carry2
        lax.fori_loop(0, K // L, group, 0)
        pltpu.sync_copy(pb, p_hbm.at[pl.ds(ebase + c * K, K)])
        pltpu.sync_copy(pb, dsh.at[dstbs[b]], add=True)

    start(0, 0)
    start(1, 1)

    def outer(c2, carry):
        for b in range(2):
            c = c2 * 2 + b
            wait(b)
            compute(c, b)

            @pl.when(c + 2 < NCH)
            def _():
                start(c + 2, b)
        return carry
    lax.fori_loop(0, NCH // 2, outer, 0)
    wait(0)
    compute(NCH - 1, 0)

    plsc.subcore_barrier()
    pltpu.sync_copy(dsh.at[pl.ds(sid * SEG, SEG)],
                    dpart_hbm.at[cid, pl.ds(sid * SEG, SEG)])


_edge1 = functools.partial(
    pl.kernel,
    out_type=[jax.ShapeDtypeStruct((E,), jnp.float32),
              jax.ShapeDtypeStruct((NC, NPAD), jnp.float32)],
    mesh=_mesh,
    scratch_types=[
        pltpu.VMEM((K, H), jnp.float32),
        pltpu.VMEM((K, H), jnp.float32),
        pltpu.VMEM((K, H), jnp.float32),
        pltpu.VMEM((K, H), jnp.float32),
        pltpu.VMEM((EW,), jnp.int32),
        pltpu.VMEM((EW,), jnp.int32),
        pltpu.VMEM((EW,), jnp.float32),
        pltpu.VMEM((K,), jnp.int32),
        pltpu.VMEM((K,), jnp.int32),
        pltpu.VMEM((K,), jnp.int32),
        pltpu.VMEM((K,), jnp.int32),
        pltpu.VMEM((K,), jnp.float32),
        pltpu.VMEM((H,), jnp.float32),
        pltpu.VMEM((SEG,), jnp.float32),
        pltpu.VMEM_SHARED((NPAD,), jnp.float32),
        pltpu.SemaphoreType.DMA,
        pltpu.SemaphoreType.DMA,
        pltpu.SemaphoreType.DMA,
        pltpu.SemaphoreType.DMA,
    ],
)(_edge1_body)


def _dsum_body(d_ref, o_ref):
    o_ref[...] = d_ref[0] + d_ref[1]


_dsum = pl.pallas_call(
    _dsum_body,
    grid=(1,),
    in_specs=[pl.BlockSpec((NC, 8, NPAD // 8), lambda i: (0, 0, 0))],
    out_specs=pl.BlockSpec((8, NPAD // 8), lambda i: (0, 0)),
    out_shape=jax.ShapeDtypeStruct((8, NPAD // 8), jnp.float32),
)


def _edge2_body(hsrc_hbm, src_hbm, dst_hbm, p_hbm, den_hbm,
                attn_hbm, hg_hbm,
                rowbuf, srcall, dstall, pall, srcb, dstb,
                dch, attnb, hg_sh):
    cid = lax.axis_index("c")
    sid = lax.axis_index("s")
    wid = cid * NS + sid
    ebase = wid * EW
    pltpu.sync_copy(src_hbm.at[pl.ds(ebase, EW)], srcall)
    pltpu.sync_copy(dst_hbm.at[pl.ds(ebase, EW)], dstall)
    pltpu.sync_copy(p_hbm.at[pl.ds(ebase, EW)], pall)

    def zrow(r, c):
        for j in range(H // L):
            rowbuf[r, pl.ds(j * L, L)] = jnp.zeros((L,), jnp.float32)
        return c
    lax.fori_loop(0, K, zrow, 0)
    for i in range(SEG // K):
        pltpu.sync_copy(rowbuf, hg_sh.at[pl.ds(sid * SEG + i * K, K)])
    plsc.subcore_barrier()

    def chunk(c, carry):
        _copy_chunk(srcall, c * K, srcb)
        _copy_chunk(dstall, c * K, dstb)
        pltpu.sync_copy(hsrc_hbm.at[srcb], rowbuf)
        pltpu.sync_copy(den_hbm.at[dstb], dch)

        def group(g, carry2):
            gbase = g * L
            attnv = pall[pl.ds(c * K + gbase, L)] / dch[pl.ds(gbase, L)]
            pall[pl.ds(c * K + gbase, L)] = attnv

            for e in range(L):
                row = gbase + e
                av = attnv[e]

                def jbody(jo, c3, _row=row, _av=av):
                    rowbuf[_row, pl.ds(jo * L, L)] = (
                        rowbuf[_row, pl.ds(jo * L, L)] * _av)
                    return c3
                lax.fori_loop(0, H // L, jbody, 0, unroll=8)
            return carry2
        lax.fori_loop(0, K // L, group, 0)
        pltpu.sync_copy(rowbuf, hg_sh.at[dstb], add=True)
        return carry
    lax.fori_loop(0, NCH, chunk, 0)
    pltpu.sync_copy(pall, attn_hbm.at[pl.ds(ebase, EW)])

    plsc.subcore_barrier()
    pltpu.sync_copy(hg_sh.at[pl.ds(sid * SEG, SEG)],
                    hg_hbm.at[cid, pl.ds(sid * SEG, SEG)])


_edge2 = functools.partial(
    pl.kernel,
    out_type=[jax.ShapeDtypeStruct((E,), jnp.float32),
              jax.ShapeDtypeStruct((NC, NPAD, H), jnp.float32)],
    mesh=_mesh,
    scratch_types=[
        pltpu.VMEM((K, H), jnp.float32),
        pltpu.VMEM((EW,), jnp.int32),
        pltpu.VMEM((EW,), jnp.int32),
        pltpu.VMEM((EW,), jnp.float32),
        pltpu.VMEM((K,), jnp.int32),
        pltpu.VMEM((K,), jnp.int32),
        pltpu.VMEM((K,), jnp.float32),
        pltpu.VMEM((K,), jnp.float32),
        pltpu.VMEM_SHARED((NPAD, H), jnp.float32),
    ],
)(_edge2_body)


def _post_body(hd_ref, g0_ref, g1_ref, wt_ref, wb_ref, b_ref, gam_ref,
               bet_ref, o_ref):
    hd = hd_ref[...]
    hg = g0_ref[...] + g1_ref[...]
    proj = (jnp.dot(hd, wt_ref[...], preferred_element_type=jnp.float32)
            + jnp.dot(hg, wb_ref[...], preferred_element_type=jnp.float32)
            + b_ref[...])
    x = hd + proj
    mu = jnp.mean(x, axis=-1, keepdims=True)
    xc = x - mu
    var = jnp.mean(xc * xc, axis=-1, keepdims=True)
    xn = xc * lax.rsqrt(var + 1e-5)
    o_ref[...] = xn * gam_ref[...] + bet_ref[...]


_post = pl.pallas_call(
    _post_body,
    grid=(N // BN,),
    in_specs=[pl.BlockSpec((BN, D), lambda i: (i, 0)),
              pl.BlockSpec((BN, H), lambda i: (i, 0)),
              pl.BlockSpec((BN, H), lambda i: (i, 0)),
              pl.BlockSpec((D, D), lambda i: (0, 0)),
              pl.BlockSpec((H, D), lambda i: (0, 0)),
              pl.BlockSpec((1, D), lambda i: (0, 0)),
              pl.BlockSpec((1, D), lambda i: (0, 0)),
              pl.BlockSpec((1, D), lambda i: (0, 0))],
    out_specs=pl.BlockSpec((BN, D), lambda i: (i, 0)),
    out_shape=jax.ShapeDtypeStruct((N, D), jnp.float32),
)


def kernel(h_src, h_dst, session_embedding, edge_index, edge_weight,
           W_attn1, W_attn2, W_out, b_out, ln_gamma, ln_beta):
    src = edge_index[0]
    dst = edge_index[1]
    w2 = W_attn2[:, 0]
    A, B = _pre(h_src, session_embedding, W_attn1[:D], W_attn1[D:])
    p, dpart = _edge1(A, B, src, dst, edge_weight, w2)
    den = _dsum(dpart.reshape(NC, 8, NPAD // 8)).reshape(NPAD)
    attn, hg = _edge2(h_src, src, dst, p, den)
    out = _post(h_dst, hg[0, :N], hg[1, :N], W_out[:D], W_out[D:],
                b_out.reshape(1, D), ln_gamma.reshape(1, D),
                ln_beta.reshape(1, D))
    return out, attn[:, None]


# speedup vs baseline: 1.1888x; 1.0170x over previous
import functools

import jax
import jax.numpy as jnp
from jax import lax
from jax.experimental import pallas as pl
from jax.experimental.pallas import tpu as pltpu
from jax.experimental.pallas import tpu_sc as plsc

N = 10000
E = 320000
D = 128
H = 128

NC = 2
NS = 16
L = 16
NW = NC * NS
EW = E // NW
K = 80
NCH = EW // K
NPAD = 10240
SEG = NPAD // NS

_mesh = plsc.VectorSubcoreMesh(core_axis_name="c", subcore_axis_name="s",
                               num_cores=NC, num_subcores=NS)


def _bitrev_perm():
    lane = lax.iota(jnp.int32, L)
    return (((lane & 1) << 3) | ((lane & 2) << 1)
            | ((lane & 4) >> 1) | ((lane & 8) >> 3))


def _merge_tree(accs, lane):
    cur = accs
    for s in (8, 4, 2, 1):
        perm = jnp.bitwise_xor(lane, s)
        nxt = []
        for i in range(len(cur) // 2):
            a, b = cur[2 * i], cur[2 * i + 1]
            pa = a + a[perm]
            pb = b + b[perm]
            nxt.append(jnp.where((lane & s) == 0, pa, pb))
        cur = nxt
    return cur[0][_bitrev_perm()]


def _copy_chunk(src_ref, src_off, dst_ref):
    for i in range(K // L):
        dst_ref[pl.ds(i * L, L)] = src_ref[pl.ds(src_off + i * L, L)]


BN = 1000


def _pre_body(h_ref, s_ref, w1t_ref, w1b_ref, a_ref, b_ref):
    a_ref[...] = jnp.dot(h_ref[...], w1t_ref[...],
                         preferred_element_type=jnp.float32)
    b_ref[...] = jnp.dot(s_ref[...], w1b_ref[...],
                         preferred_element_type=jnp.float32)


_pre = pl.pallas_call(
    _pre_body,
    grid=(N // BN,),
    in_specs=[pl.BlockSpec((BN, D), lambda i: (i, 0)),
              pl.BlockSpec((BN, D), lambda i: (i, 0)),
              pl.BlockSpec((D, H), lambda i: (0, 0)),
              pl.BlockSpec((D, H), lambda i: (0, 0))],
    out_specs=[pl.BlockSpec((BN, H), lambda i: (i, 0)),
               pl.BlockSpec((BN, H), lambda i: (i, 0))],
    out_shape=[jax.ShapeDtypeStruct((N, H), jnp.float32),
               jax.ShapeDtypeStruct((N, H), jnp.float32)],
)


def _edge1_body(a_hbm, b_hbm, src_hbm, dst_hbm, ew_hbm, w2_hbm,
                p_hbm, dpart_hbm,
                bufA0, bufA1, bufB0, bufB1, srcall, dstall, ewall, pall,
                dstb0, dstb1,
                w2v, zb, dsh, sA0, sA1, sB0, sB1):
    cid = lax.axis_index("c")
    sid = lax.axis_index("s")
    wid = cid * NS + sid
    ebase = wid * EW
    pltpu.sync_copy(w2_hbm, w2v)
    pltpu.sync_copy(src_hbm.at[pl.ds(ebase, EW)], srcall)
    pltpu.sync_copy(dst_hbm.at[pl.ds(ebase, EW)], dstall)
    pltpu.sync_copy(ew_hbm.at[pl.ds(ebase, EW)], ewall)

    def zb_init(i, c):
        zb[pl.ds(i * L, L)] = jnp.zeros((L,), jnp.float32)
        return c
    lax.fori_loop(0, SEG // L, zb_init, 0)
    pltpu.sync_copy(zb, dsh.at[pl.ds(sid * SEG, SEG)])
    plsc.subcore_barrier()

    lane = lax.iota(jnp.int32, L)
    w2s = [w2v[pl.ds(j * L, L)] for j in range(H // L)]
    bufsA = (bufA0, bufA1)
    bufsB = (bufB0, bufB1)
    dstbs = (dstb0, dstb1)
    semsA = (sA0, sA1)
    semsB = (sB0, sB1)

    def start(c, b):
        _copy_chunk(dstall, c * K, dstbs[b])
        pltpu.async_copy(a_hbm.at[srcall.at[pl.ds(c * K, K)]], bufsA[b],
                         semsA[b])
        pltpu.async_copy(b_hbm.at[dstbs[b]], bufsB[b], semsB[b])

    def wait(c, b):
        pltpu.make_async_copy(a_hbm.at[srcall.at[pl.ds(c * K, K)]], bufsA[b],
                              semsA[b]).wait()
        pltpu.make_async_copy(b_hbm.at[dstbs[b]], bufsB[b], semsB[b]).wait()

    perms = [jnp.bitwise_xor(lane, s) for s in (8, 4, 2, 1)]

    def compute(c, b):
        bufA, bufB = bufsA[b], bufsB[b]

        def group(g, carry2):
            gbase = g * L

            def ebody(e, rawv):
                row = gbase + e

                acc = jnp.zeros((L,), jnp.float32)
                for jo in range(H // L):
                    x = (bufA[row, pl.ds(jo * L, L)]
                         + bufB[row, pl.ds(jo * L, L)])
                    e2 = jnp.exp(x + x)
                    t = 1.0 - 2.0 / (e2 + 1.0)
                    acc = acc + t * w2s[jo]
                for pm in perms:
                    acc = acc + acc[pm]
                return jnp.where(lane == e, acc, rawv)

            rawv = lax.fori_loop(0, L, ebody, jnp.zeros((L,), jnp.float32),
                                 unroll=2)
            pv = jnp.exp(rawv * ewall[pl.ds(c * K + gbase, L)])
            pall[pl.ds(c * K + gbase, L)] = pv
            return carry2
        lax.fori_loop(0, K // L, group, 0)
        pltpu.sync_copy(pall.at[pl.ds(c * K, K)], dsh.at[dstbs[b]], add=True)

    start(0, 0)
    start(1, 1)

    def outer(c2, carry):
        for b in range(2):
            c = c2 * 2 + b
            wait(c, b)
            compute(c, b)

            @pl.when(c + 2 < NCH)
            def _():
                start(c + 2, b)
        return carry
    lax.fori_loop(0, NCH // 2, outer, 0)
    wait(NCH - 1, 0)
    compute(NCH - 1, 0)
    pltpu.sync_copy(pall, p_hbm.at[pl.ds(ebase, EW)])

    plsc.subcore_barrier()
    pltpu.sync_copy(dsh.at[pl.ds(sid * SEG, SEG)],
                    dpart_hbm.at[cid, pl.ds(sid * SEG, SEG)])


_edge1 = functools.partial(
    pl.kernel,
    out_type=[jax.ShapeDtypeStruct((E,), jnp.float32),
              jax.ShapeDtypeStruct((NC, NPAD), jnp.float32)],
    mesh=_mesh,
    scratch_types=[
        pltpu.VMEM((K, H), jnp.float32),
        pltpu.VMEM((K, H), jnp.float32),
        pltpu.VMEM((K, H), jnp.float32),
        pltpu.VMEM((K, H), jnp.float32),
        pltpu.VMEM((EW,), jnp.int32),
        pltpu.VMEM((EW,), jnp.int32),
        pltpu.VMEM((EW,), jnp.float32),
        pltpu.VMEM((EW,), jnp.float32),
        pltpu.VMEM((K,), jnp.int32),
        pltpu.VMEM((K,), jnp.int32),
        pltpu.VMEM((H,), jnp.float32),
        pltpu.VMEM((SEG,), jnp.float32),
        pltpu.VMEM_SHARED((NPAD,), jnp.float32),
        pltpu.SemaphoreType.DMA,
        pltpu.SemaphoreType.DMA,
        pltpu.SemaphoreType.DMA,
        pltpu.SemaphoreType.DMA,
    ],
)(_edge1_body)


def _dsum_body(d_ref, o_ref):
    o_ref[...] = d_ref[0] + d_ref[1]


_dsum = pl.pallas_call(
    _dsum_body,
    grid=(1,),
    in_specs=[pl.BlockSpec((NC, 8, NPAD // 8), lambda i: (0, 0, 0))],
    out_specs=pl.BlockSpec((8, NPAD // 8), lambda i: (0, 0)),
    out_shape=jax.ShapeDtypeStruct((8, NPAD // 8), jnp.float32),
)


def _edge2_body(hsrc_hbm, src_hbm, dst_hbm, p_hbm, den_hbm,
                attn_hbm, hg_hbm,
                rowbuf, srcall, dstall, pall, dstb,
                dch, attnb, hg_sh):
    cid = lax.axis_index("c")
    sid = lax.axis_index("s")
    wid = cid * NS + sid
    ebase = wid * EW
    pltpu.sync_copy(src_hbm.at[pl.ds(ebase, EW)], srcall)
    pltpu.sync_copy(dst_hbm.at[pl.ds(ebase, EW)], dstall)
    pltpu.sync_copy(p_hbm.at[pl.ds(ebase, EW)], pall)

    def zrow(r, c):
        for j in range(H // L):
            rowbuf[r, pl.ds(j * L, L)] = jnp.zeros((L,), jnp.float32)
        return c
    lax.fori_loop(0, K, zrow, 0)
    for i in range(SEG // K):
        pltpu.sync_copy(rowbuf, hg_sh.at[pl.ds(sid * SEG + i * K, K)])
    plsc.subcore_barrier()

    def chunk(c, carry):
        _copy_chunk(dstall, c * K, dstb)
        pltpu.sync_copy(hsrc_hbm.at[srcall.at[pl.ds(c * K, K)]], rowbuf)
        pltpu.sync_copy(den_hbm.at[dstb], dch)

        def group(g, carry2):
            gbase = g * L
            attnv = pall[pl.ds(c * K + gbase, L)] / dch[pl.ds(gbase, L)]
            pall[pl.ds(c * K + gbase, L)] = attnv

            for e in range(L):
                row = gbase + e
                av = attnv[e]

                def jbody(jo, c3, _row=row, _av=av):
                    rowbuf[_row, pl.ds(jo * L, L)] = (
                        rowbuf[_row, pl.ds(jo * L, L)] * _av)
                    return c3
                lax.fori_loop(0, H // L, jbody, 0, unroll=8)
            return carry2
        lax.fori_loop(0, K // L, group, 0)
        pltpu.sync_copy(rowbuf, hg_sh.at[dstb], add=True)
        return carry
    lax.fori_loop(0, NCH, chunk, 0)
    pltpu.sync_copy(pall, attn_hbm.at[pl.ds(ebase, EW)])

    plsc.subcore_barrier()
    pltpu.sync_copy(hg_sh.at[pl.ds(sid * SEG, SEG)],
                    hg_hbm.at[cid, pl.ds(sid * SEG, SEG)])


_edge2 = functools.partial(
    pl.kernel,
    out_type=[jax.ShapeDtypeStruct((E,), jnp.float32),
              jax.ShapeDtypeStruct((NC, NPAD, H), jnp.float32)],
    mesh=_mesh,
    scratch_types=[
        pltpu.VMEM((K, H), jnp.float32),
        pltpu.VMEM((EW,), jnp.int32),
        pltpu.VMEM((EW,), jnp.int32),
        pltpu.VMEM((EW,), jnp.float32),
        pltpu.VMEM((K,), jnp.int32),
        pltpu.VMEM((K,), jnp.float32),
        pltpu.VMEM((K,), jnp.float32),
        pltpu.VMEM_SHARED((NPAD, H), jnp.float32),
    ],
)(_edge2_body)


def _post_body(hd_ref, g0_ref, g1_ref, wt_ref, wb_ref, b_ref, gam_ref,
               bet_ref, o_ref):
    hd = hd_ref[...]
    hg = g0_ref[...] + g1_ref[...]
    proj = (jnp.dot(hd, wt_ref[...], preferred_element_type=jnp.float32)
            + jnp.dot(hg, wb_ref[...], preferred_element_type=jnp.float32)
            + b_ref[...])
    x = hd + proj
    mu = jnp.mean(x, axis=-1, keepdims=True)
    xc = x - mu
    var = jnp.mean(xc * xc, axis=-1, keepdims=True)
    xn = xc * lax.rsqrt(var + 1e-5)
    o_ref[...] = xn * gam_ref[...] + bet_ref[...]


_post = pl.pallas_call(
    _post_body,
    grid=(N // BN,),
    in_specs=[pl.BlockSpec((BN, D), lambda i: (i, 0)),
              pl.BlockSpec((BN, H), lambda i: (i, 0)),
              pl.BlockSpec((BN, H), lambda i: (i, 0)),
              pl.BlockSpec((D, D), lambda i: (0, 0)),
              pl.BlockSpec((H, D), lambda i: (0, 0)),
              pl.BlockSpec((1, D), lambda i: (0, 0)),
              pl.BlockSpec((1, D), lambda i: (0, 0)),
              pl.BlockSpec((1, D), lambda i: (0, 0))],
    out_specs=pl.BlockSpec((BN, D), lambda i: (i, 0)),
    out_shape=jax.ShapeDtypeStruct((N, D), jnp.float32),
)


def kernel(h_src, h_dst, session_embedding, edge_index, edge_weight,
           W_attn1, W_attn2, W_out, b_out, ln_gamma, ln_beta):
    src = edge_index[0]
    dst = edge_index[1]
    w2 = W_attn2[:, 0]
    A, B = _pre(h_src, session_embedding, W_attn1[:D], W_attn1[D:])
    p, dpart = _edge1(A, B, src, dst, edge_weight, w2)
    den = _dsum(dpart.reshape(NC, 8, NPAD // 8)).reshape(NPAD)
    attn, hg = _edge2(h_src, src, dst, p, den)
    out = _post(h_dst, hg[0, :N], hg[1, :N], W_out[:D], W_out[D:],
                b_out.reshape(1, D), ln_gamma.reshape(1, D),
                ln_beta.reshape(1, D))
    return out, attn[:, None]


# speedup vs baseline: 1.1919x; 1.0026x over previous
import functools

import jax
import jax.numpy as jnp
from jax import lax
from jax.experimental import pallas as pl
from jax.experimental.pallas import tpu as pltpu
from jax.experimental.pallas import tpu_sc as plsc

N = 10000
E = 320000
D = 128
H = 128

NC = 2
NS = 16
L = 16
NW = NC * NS
EW = E // NW
K = 80
NCH = EW // K
NPAD = 10240
SEG = NPAD // NS

_mesh = plsc.VectorSubcoreMesh(core_axis_name="c", subcore_axis_name="s",
                               num_cores=NC, num_subcores=NS)


def _bitrev_perm():
    lane = lax.iota(jnp.int32, L)
    return (((lane & 1) << 3) | ((lane & 2) << 1)
            | ((lane & 4) >> 1) | ((lane & 8) >> 3))


def _merge_tree(accs, lane):
    cur = accs
    for s in (8, 4, 2, 1):
        perm = jnp.bitwise_xor(lane, s)
        nxt = []
        for i in range(len(cur) // 2):
            a, b = cur[2 * i], cur[2 * i + 1]
            pa = a + a[perm]
            pb = b + b[perm]
            nxt.append(jnp.where((lane & s) == 0, pa, pb))
        cur = nxt
    return cur[0][_bitrev_perm()]


def _copy_chunk(src_ref, src_off, dst_ref):
    for i in range(K // L):
        dst_ref[pl.ds(i * L, L)] = src_ref[pl.ds(src_off + i * L, L)]


BN = 1000


def _pre_body(h_ref, s_ref, w1t_ref, w1b_ref, a_ref, b_ref):
    a_ref[...] = jnp.dot(h_ref[...], w1t_ref[...],
                         preferred_element_type=jnp.float32)
    b_ref[...] = jnp.dot(s_ref[...], w1b_ref[...],
                         preferred_element_type=jnp.float32)


_pre = pl.pallas_call(
    _pre_body,
    grid=(N // BN,),
    in_specs=[pl.BlockSpec((BN, D), lambda i: (i, 0)),
              pl.BlockSpec((BN, D), lambda i: (i, 0)),
              pl.BlockSpec((D, H), lambda i: (0, 0)),
              pl.BlockSpec((D, H), lambda i: (0, 0))],
    out_specs=[pl.BlockSpec((BN, H), lambda i: (i, 0)),
               pl.BlockSpec((BN, H), lambda i: (i, 0))],
    out_shape=[jax.ShapeDtypeStruct((N, H), jnp.float32),
               jax.ShapeDtypeStruct((N, H), jnp.float32)],
)


K1 = 128
NCH1 = EW // K1
TAIL = EW - NCH1 * K1


def _edge1_body(a_hbm, b_hbm, src_hbm, dst_hbm, ew_hbm, w2_hbm,
                p_hbm, dpart_hbm,
                bufA0, bufA1, bufB0, bufB1, srcall, dstall, ewall, pall,
                dstb0, dstb1, dstbT,
                w2v, zb, dsh, sA0, sA1, sB0, sB1):
    cid = lax.axis_index("c")
    sid = lax.axis_index("s")
    wid = cid * NS + sid
    ebase = wid * EW
    pltpu.sync_copy(w2_hbm, w2v)
    pltpu.sync_copy(src_hbm.at[pl.ds(ebase, EW)], srcall)
    pltpu.sync_copy(dst_hbm.at[pl.ds(ebase, EW)], dstall)
    pltpu.sync_copy(ew_hbm.at[pl.ds(ebase, EW)], ewall)

    def zb_init(i, c):
        zb[pl.ds(i * L, L)] = jnp.zeros((L,), jnp.float32)
        return c
    lax.fori_loop(0, SEG // L, zb_init, 0)
    pltpu.sync_copy(zb, dsh.at[pl.ds(sid * SEG, SEG)])
    plsc.subcore_barrier()

    lane = lax.iota(jnp.int32, L)
    w2s = [w2v[pl.ds(j * L, L)] for j in range(H // L)]
    bufsA = (bufA0, bufA1)
    bufsB = (bufB0, bufB1)
    dstbs = (dstb0, dstb1)
    semsA = (sA0, sA1)
    semsB = (sB0, sB1)
    perms = [jnp.bitwise_xor(lane, s) for s in (8, 4, 2, 1)]

    def start(c, b):
        off = c * K1
        for i in range(K1 // L):
            dstbs[b][pl.ds(i * L, L)] = dstall[pl.ds(off + i * L, L)]
        pltpu.async_copy(a_hbm.at[srcall.at[pl.ds(off, K1)]], bufsA[b],
                         semsA[b])
        pltpu.async_copy(b_hbm.at[dstbs[b]], bufsB[b], semsB[b])

    def wait(c, b):
        off = c * K1
        pltpu.make_async_copy(a_hbm.at[srcall.at[pl.ds(off, K1)]], bufsA[b],
                              semsA[b]).wait()
        pltpu.make_async_copy(b_hbm.at[dstbs[b]], bufsB[b], semsB[b]).wait()

    def one_group(bufA, bufB, gbase, poff):
        def ebody(e, rawv):
            row = gbase + e
            acc = jnp.zeros((L,), jnp.float32)
            for jo in range(H // L):
                x = (bufA[row, pl.ds(jo * L, L)]
                     + bufB[row, pl.ds(jo * L, L)])
                e2 = jnp.exp(x + x)
                t = 1.0 - 2.0 / (e2 + 1.0)
                acc = acc + t * w2s[jo]
            for pm in perms:
                acc = acc + acc[pm]
            return jnp.where(lane == e, acc, rawv)

        rawv = lax.fori_loop(0, L, ebody, jnp.zeros((L,), jnp.float32),
                             unroll=2)
        pv = jnp.exp(rawv * ewall[pl.ds(poff, L)])
        pall[pl.ds(poff, L)] = pv

    def compute(c, b):
        bufA, bufB = bufsA[b], bufsB[b]

        def group(g, carry2):
            one_group(bufA, bufB, g * L, c * K1 + g * L)
            return carry2
        lax.fori_loop(0, K1 // L, group, 0)
        pltpu.sync_copy(pall.at[pl.ds(c * K1, K1)], dsh.at[dstbs[b]],
                        add=True)

    start(0, 0)
    start(1, 1)

    def outer(c2, carry):
        for b in range(2):
            c = c2 * 2 + b
            wait(c, b)
            compute(c, b)

            @pl.when(c + 2 < NCH1)
            def _():
                start(c + 2, b)
        return carry
    lax.fori_loop(0, NCH1 // 2, outer, 0)

    toff = NCH1 * K1
    dstbT[pl.ds(0, L)] = dstall[pl.ds(toff, L)]
    pltpu.sync_copy(a_hbm.at[srcall.at[pl.ds(toff, TAIL)]],
                    bufA0.at[pl.ds(0, TAIL)])
    pltpu.sync_copy(b_hbm.at[dstbT], bufB0.at[pl.ds(0, TAIL)])
    one_group(bufA0, bufB0, 0, toff)
    pltpu.sync_copy(pall.at[pl.ds(toff, TAIL)], dsh.at[dstbT], add=True)
    pltpu.sync_copy(pall, p_hbm.at[pl.ds(ebase, EW)])

    plsc.subcore_barrier()
    pltpu.sync_copy(dsh.at[pl.ds(sid * SEG, SEG)],
                    dpart_hbm.at[cid, pl.ds(sid * SEG, SEG)])


_edge1 = functools.partial(
    pl.kernel,
    out_type=[jax.ShapeDtypeStruct((E,), jnp.float32),
              jax.ShapeDtypeStruct((NC, NPAD), jnp.float32)],
    mesh=_mesh,
    scratch_types=[
        pltpu.VMEM((K1, H), jnp.float32),
        pltpu.VMEM((K1, H), jnp.float32),
        pltpu.VMEM((K1, H), jnp.float32),
        pltpu.VMEM((K1, H), jnp.float32),
        pltpu.VMEM((EW,), jnp.int32),
        pltpu.VMEM((EW,), jnp.int32),
        pltpu.VMEM((EW,), jnp.float32),
        pltpu.VMEM((EW,), jnp.float32),
        pltpu.VMEM((K1,), jnp.int32),
        pltpu.VMEM((K1,), jnp.int32),
        pltpu.VMEM((TAIL,), jnp.int32),
        pltpu.VMEM((H,), jnp.float32),
        pltpu.VMEM((SEG,), jnp.float32),
        pltpu.VMEM_SHARED((NPAD,), jnp.float32),
        pltpu.SemaphoreType.DMA,
        pltpu.SemaphoreType.DMA,
        pltpu.SemaphoreType.DMA,
        pltpu.SemaphoreType.DMA,
    ],
)(_edge1_body)


def _dsum_body(d_ref, o_ref):
    o_ref[...] = d_ref[0] + d_ref[1]


_dsum = pl.pallas_call(
    _dsum_body,
    grid=(1,),
    in_specs=[pl.BlockSpec((NC, 8, NPAD // 8), lambda i: (0, 0, 0))],
    out_specs=pl.BlockSpec((8, NPAD // 8), lambda i: (0, 0)),
    out_shape=jax.ShapeDtypeStruct((8, NPAD // 8), jnp.float32),
)


def _edge2_body(hsrc_hbm, src_hbm, dst_hbm, p_hbm, den_hbm,
                attn_hbm, hg_hbm,
                rowbuf, srcall, dstall, pall, dstb,
                dch, attnb, hg_sh):
    cid = lax.axis_index("c")
    sid = lax.axis_index("s")
    wid = cid * NS + sid
    ebase = wid * EW
    pltpu.sync_copy(src_hbm.at[pl.ds(ebase, EW)], srcall)
    pltpu.sync_copy(dst_hbm.at[pl.ds(ebase, EW)], dstall)
    pltpu.sync_copy(p_hbm.at[pl.ds(ebase, EW)], pall)

    def zrow(r, c):
        for j in range(H // L):
            rowbuf[r, pl.ds(j * L, L)] = jnp.zeros((L,), jnp.float32)
        return c
    lax.fori_loop(0, K, zrow, 0)
    for i in range(SEG // K):
        pltpu.sync_copy(rowbuf, hg_sh.at[pl.ds(sid * SEG + i * K, K)])
    plsc.subcore_barrier()

    def chunk(c, carry):
        _copy_chunk(dstall, c * K, dstb)
        pltpu.sync_copy(hsrc_hbm.at[srcall.at[pl.ds(c * K, K)]], rowbuf)
        pltpu.sync_copy(den_hbm.at[dstb], dch)

        def group(g, carry2):
            gbase = g * L
            attnv = pall[pl.ds(c * K + gbase, L)] / dch[pl.ds(gbase, L)]
            pall[pl.ds(c * K + gbase, L)] = attnv

            for e in range(L):
                row = gbase + e
                av = attnv[e]

                def jbody(jo, c3, _row=row, _av=av):
                    rowbuf[_row, pl.ds(jo * L, L)] = (
                        rowbuf[_row, pl.ds(jo * L, L)] * _av)
                    return c3
                lax.fori_loop(0, H // L, jbody, 0, unroll=8)
            return carry2
        lax.fori_loop(0, K // L, group, 0)
        pltpu.sync_copy(rowbuf, hg_sh.at[dstb], add=True)
        return carry
    lax.fori_loop(0, NCH, chunk, 0)
    pltpu.sync_copy(pall, attn_hbm.at[pl.ds(ebase, EW)])

    plsc.subcore_barrier()
    pltpu.sync_copy(hg_sh.at[pl.ds(sid * SEG, SEG)],
                    hg_hbm.at[cid, pl.ds(sid * SEG, SEG)])


_edge2 = functools.partial(
    pl.kernel,
    out_type=[jax.ShapeDtypeStruct((E,), jnp.float32),
              jax.ShapeDtypeStruct((NC, NPAD, H), jnp.float32)],
    mesh=_mesh,
    scratch_types=[
        pltpu.VMEM((K, H), jnp.float32),
        pltpu.VMEM((EW,), jnp.int32),
        pltpu.VMEM((EW,), jnp.int32),
        pltpu.VMEM((EW,), jnp.float32),
        pltpu.VMEM((K,), jnp.int32),
        pltpu.VMEM((K,), jnp.float32),
        pltpu.VMEM((K,), jnp.float32),
        pltpu.VMEM_SHARED((NPAD, H), jnp.float32),
    ],
)(_edge2_body)


def _post_body(hd_ref, g0_ref, g1_ref, wt_ref, wb_ref, b_ref, gam_ref,
               bet_ref, o_ref):
    hd = hd_ref[...]
    hg = g0_ref[...] + g1_ref[...]
    proj = (jnp.dot(hd, wt_ref[...], preferred_element_type=jnp.float32)
            + jnp.dot(hg, wb_ref[...], preferred_element_type=jnp.float32)
            + b_ref[...])
    x = hd + proj
    mu = jnp.mean(x, axis=-1, keepdims=True)
    xc = x - mu
    var = jnp.mean(xc * xc, axis=-1, keepdims=True)
    xn = xc * lax.rsqrt(var + 1e-5)
    o_ref[...] = xn * gam_ref[...] + bet_ref[...]


_post = pl.pallas_call(
    _post_body,
    grid=(N // BN,),
    in_specs=[pl.BlockSpec((BN, D), lambda i: (i, 0)),
              pl.BlockSpec((BN, H), lambda i: (i, 0)),
              pl.BlockSpec((BN, H), lambda i: (i, 0)),
              pl.BlockSpec((D, D), lambda i: (0, 0)),
              pl.BlockSpec((H, D), lambda i: (0, 0)),
              pl.BlockSpec((1, D), lambda i: (0, 0)),
              pl.BlockSpec((1, D), lambda i: (0, 0)),
              pl.BlockSpec((1, D), lambda i: (0, 0))],
    out_specs=pl.BlockSpec((BN, D), lambda i: (i, 0)),
    out_shape=jax.ShapeDtypeStruct((N, D), jnp.float32),
)


def kernel(h_src, h_dst, session_embedding, edge_index, edge_weight,
           W_attn1, W_attn2, W_out, b_out, ln_gamma, ln_beta):
    src = edge_index[0]
    dst = edge_index[1]
    w2 = W_attn2[:, 0]
    A, B = _pre(h_src, session_embedding, W_attn1[:D], W_attn1[D:])
    p, dpart = _edge1(A, B, src, dst, edge_weight, w2)
    den = _dsum(dpart.reshape(NC, 8, NPAD // 8)).reshape(NPAD)
    attn, hg = _edge2(h_src, src, dst, p, den)
    out = _post(h_dst, hg[0, :N], hg[1, :N], W_out[:D], W_out[D:],
                b_out.reshape(1, D), ln_gamma.reshape(1, D),
                ln_beta.reshape(1, D))
    return out, attn[:, None]
